# Initial kernel scaffold; baseline (speedup 1.0000x reference)
import functools

import jax
import jax.numpy as jnp
from jax import lax
from jax.experimental import pallas as pl
from jax.experimental.pallas import tpu as pltpu
from jax.experimental.pallas import tpu_sc as plsc

_N = 10000
_E = 320000
_D = 128
_H = 64
_CH = 128
_NT = 16
_RPT = _N // _NT
_ROWBLK = 1000


def _sc_mesh():
    return plsc.VectorSubcoreMesh(core_axis_name="c", subcore_axis_name="s")


def _zero_stage(stage_v):
    zero16 = jnp.zeros((16,), jnp.float32)

    def zrow(r, carry):
        for j in range(_H // 16):
            stage_v[r, pl.ds(j * 16, 16)] = zero16
        return carry

    lax.fori_loop(0, _RPT, zrow, 0)


def _edge_sweep(cbase, ccount, src_h, dst_h, w_h, t_h, si_v, di_v, w_v,
                rows_v, acc_sh, sem):
    def body(k_i, carry):
        off = (cbase + k_i) * _CH
        pltpu.sync_copy(src_h.at[pl.ds(off, _CH)], si_v)
        pltpu.sync_copy(dst_h.at[pl.ds(off, _CH)], di_v)
        pltpu.sync_copy(w_h.at[pl.ds(off, _CH)], w_v)
        pltpu.async_copy(t_h.at[si_v], rows_v, sem).wait()

        def scale(i, c2):
            wv = w_v[i]
            for j in range(_H // 16):
                sl = pl.ds(j * 16, 16)
                rows_v[i, sl] = rows_v[i, sl] * wv
            return c2

        lax.fori_loop(0, _CH, scale, 0)
        pltpu.sync_copy(rows_v, acc_sh.at[di_v], add=True)
        return carry

    lax.fori_loop(0, ccount, body, 0)


def _spmm_hop1(src, dst, w, t0, t1):
    nchunks = _E // _CH
    per, extra = nchunks // _NT, nchunks % _NT

    @functools.partial(
        pl.kernel,
        mesh=_sc_mesh(),
        out_type=[jax.ShapeDtypeStruct((_N, _H), jnp.float32),
                  jax.ShapeDtypeStruct((_N, _H), jnp.float32)],
        scratch_types=[
            pltpu.VMEM((_CH,), jnp.int32),
            pltpu.VMEM((_CH,), jnp.int32),
            pltpu.VMEM((_CH,), jnp.float32),
            pltpu.VMEM((_CH, _H), jnp.float32),
            pltpu.VMEM((_RPT, _H), jnp.float32),
            pltpu.VMEM_SHARED((_N, _H), jnp.float32),
            pltpu.SemaphoreType.DMA,
        ],
    )
    def k(src_h, dst_h, w_h, t0_h, t1_h, o0_h, o1_h,
          si_v, di_v, w_v, rows_v, stage_v, acc_sh, sem):
        c = lax.axis_index("c")
        s = lax.axis_index("s")
        _zero_stage(stage_v)
        pltpu.sync_copy(stage_v, acc_sh.at[pl.ds(s * _RPT, _RPT)])
        plsc.subcore_barrier()

        cbase = per * s + jnp.minimum(s, extra)
        ccount = per + jnp.where(s < extra, 1, 0)

        @pl.when(c == 0)
        def _():
            _edge_sweep(cbase, ccount, src_h, dst_h, w_h, t0_h,
                        si_v, di_v, w_v, rows_v, acc_sh, sem)

        @pl.when(c == 1)
        def _():
            _edge_sweep(cbase, ccount, src_h, dst_h, w_h, t1_h,
                        si_v, di_v, w_v, rows_v, acc_sh, sem)

        plsc.subcore_barrier()
        r0 = s * _RPT
        pltpu.sync_copy(acc_sh.at[pl.ds(r0, _RPT)], stage_v)

        @pl.when(c == 0)
        def _():
            pltpu.sync_copy(stage_v, o0_h.at[pl.ds(r0, _RPT)])

        @pl.when(c == 1)
        def _():
            pltpu.sync_copy(stage_v, o1_h.at[pl.ds(r0, _RPT)])

    return k(src, dst, w, t0, t1)


def _spmm_hop2(src, dst, w, t):
    nchunks_half = (_E // 2) // _CH
    per, extra = nchunks_half // _NT, nchunks_half % _NT

    @functools.partial(
        pl.kernel,
        mesh=_sc_mesh(),
        out_type=jax.ShapeDtypeStruct((2, _N, _H), jnp.float32),
        scratch_types=[
            pltpu.VMEM((_CH,), jnp.int32),
            pltpu.VMEM((_CH,), jnp.int32),
            pltpu.VMEM((_CH,), jnp.float32),
            pltpu.VMEM((_CH, _H), jnp.float32),
            pltpu.VMEM((_RPT, _H), jnp.float32),
            pltpu.VMEM_SHARED((_N, _H), jnp.float32),
            pltpu.SemaphoreType.DMA,
        ],
    )
    def k(src_h, dst_h, w_h, t_h, o_h,
          si_v, di_v, w_v, rows_v, stage_v, acc_sh, sem):
        c = lax.axis_index("c")
        s = lax.axis_index("s")
        _zero_stage(stage_v)
        pltpu.sync_copy(stage_v, acc_sh.at[pl.ds(s * _RPT, _RPT)])
        plsc.subcore_barrier()

        cbase = c * nchunks_half + per * s + jnp.minimum(s, extra)
        ccount = per + jnp.where(s < extra, 1, 0)
        _edge_sweep(cbase, ccount, src_h, dst_h, w_h, t_h,
                    si_v, di_v, w_v, rows_v, acc_sh, sem)

        plsc.subcore_barrier()
        r0 = s * _RPT
        pltpu.sync_copy(acc_sh.at[pl.ds(r0, _RPT)], stage_v)
        pltpu.sync_copy(stage_v, o_h.at[c, pl.ds(r0, _RPT)])

    return k(src, dst, w, t)


def _tc_first(x, wcat):
    def body(x_ref, w_ref, o0_ref, o1_ref):
        u = jnp.dot(x_ref[...], w_ref[...],
                    preferred_element_type=jnp.float32)
        o0_ref[...] = u[:, :_H]
        o1_ref[...] = u[:, _H:]

    return pl.pallas_call(
        body,
        grid=(_N // _ROWBLK,),
        in_specs=[pl.BlockSpec((_ROWBLK, _D), lambda i: (i, 0)),
                  pl.BlockSpec((_D, _D), lambda i: (0, 0))],
        out_specs=[pl.BlockSpec((_ROWBLK, _H), lambda i: (i, 0)),
                   pl.BlockSpec((_ROWBLK, _H), lambda i: (i, 0))],
        out_shape=[jax.ShapeDtypeStruct((_N, _H), jnp.float32),
                   jax.ShapeDtypeStruct((_N, _H), jnp.float32)],
    )(x, wcat)


def _tc_layer(keep, p0, p1, wcat, bvec):
    def body(k_ref, p0_ref, p1_ref, w_ref, b_ref, o0_ref, o1_ref):
        wl = w_ref[...]
        upper = p0_ref[...] + p1_ref[...]
        u = (jnp.dot(k_ref[...], wl[:_H, :],
                     preferred_element_type=jnp.float32)
             + jnp.dot(upper, wl[_H:, :],
                       preferred_element_type=jnp.float32)
             + jnp.dot(b_ref[...], wl, preferred_element_type=jnp.float32))
        o0_ref[...] = u[:, :_H]
        o1_ref[...] = u[:, _H:]

    return pl.pallas_call(
        body,
        grid=(_N // _ROWBLK,),
        in_specs=[pl.BlockSpec((_ROWBLK, _H), lambda i: (i, 0)),
                  pl.BlockSpec((_ROWBLK, _H), lambda i: (i, 0)),
                  pl.BlockSpec((_ROWBLK, _H), lambda i: (i, 0)),
                  pl.BlockSpec((_D, _D), lambda i: (0, 0)),
                  pl.BlockSpec((1, _D), lambda i: (0, 0))],
        out_specs=[pl.BlockSpec((_ROWBLK, _H), lambda i: (i, 0)),
                   pl.BlockSpec((_ROWBLK, _H), lambda i: (i, 0))],
        out_shape=[jax.ShapeDtypeStruct((_N, _H), jnp.float32),
                   jax.ShapeDtypeStruct((_N, _H), jnp.float32)],
    )(keep, p0, p1, wcat, bvec)


def _tc_final(keep, p0, p1, wcp, bvec, bcp):
    def body(k_ref, p0_ref, p1_ref, w_ref, b_ref, bc_ref, o_ref):
        wl = w_ref[...]
        upper = p0_ref[...] + p1_ref[...]
        o_ref[...] = (jnp.dot(k_ref[...], wl[:_H, :],
                              preferred_element_type=jnp.float32)
                      + jnp.dot(upper, wl[_H:, :],
                                preferred_element_type=jnp.float32)
                      + jnp.dot(b_ref[...], wl,
                                preferred_element_type=jnp.float32)
                      + bc_ref[...])

    return pl.pallas_call(
        body,
        grid=(_N // _ROWBLK,),
        in_specs=[pl.BlockSpec((_ROWBLK, _H), lambda i: (i, 0)),
                  pl.BlockSpec((_ROWBLK, _H), lambda i: (i, 0)),
                  pl.BlockSpec((_ROWBLK, _H), lambda i: (i, 0)),
                  pl.BlockSpec((_D, _D), lambda i: (0, 0)),
                  pl.BlockSpec((1, _D), lambda i: (0, 0)),
                  pl.BlockSpec((1, _D), lambda i: (0, 0))],
        out_specs=pl.BlockSpec((_ROWBLK, _D), lambda i: (i, 0)),
        out_shape=jax.ShapeDtypeStruct((_N, _D), jnp.float32),
    )(keep, p0, p1, wcp, bvec, bcp)


def kernel(x, edge_index, edge_weight, W, b, Wc, bc):
    src = edge_index[0]
    dst = edge_index[1]
    w = edge_weight
    nclass = Wc.shape[1]

    t0, t1 = _tc_first(x, jnp.concatenate([W[0, 0], W[0, 1]], axis=1))
    for l in range(W.shape[0]):
        keep, upper = _spmm_hop1(src, dst, w, t0, t1)
        parts = _spmm_hop2(src, dst, w, upper)
        p0, p1 = parts[0], parts[1]
        bvec = jnp.concatenate([b[l, 0], b[l, 1]])[None, :]
        if l + 1 < W.shape[0]:
            wcat = jnp.concatenate([W[l + 1, 0], W[l + 1, 1]], axis=1)
            t0, t1 = _tc_layer(keep, p0, p1, wcat, bvec)
        else:
            wcp = jnp.pad(Wc, ((0, 0), (0, _D - nclass)))
            bcp = jnp.pad(bc, (0, _D - nclass))[None, :]
            out = _tc_final(keep, p0, p1, wcp, bvec, bcp)
            return out[:, :nclass]


# speedup vs baseline: 2.5678x; 2.5678x over previous
import functools

import jax
import jax.numpy as jnp
from jax import lax
from jax.experimental import pallas as pl
from jax.experimental.pallas import tpu as pltpu
from jax.experimental.pallas import tpu_sc as plsc

_N = 10000
_E = 320000
_D = 128
_H = 64
_CH = 128
_NT = 16
_NP = 10240
_RPT = _NP // _NT
_ROWBLK = 1000


def _sc_mesh():
    return plsc.VectorSubcoreMesh(core_axis_name="c", subcore_axis_name="s")


def _zero_stage(stage_v):
    zero16 = jnp.zeros((16,), jnp.float32)

    def zrow(r, carry):
        for j in range(_H // 16):
            stage_v[r, pl.ds(j * 16, 16)] = zero16
        return carry

    lax.fori_loop(0, _RPT, zrow, 0)


def _edge_sweep(cbase, ccount, src_h, dst_h, w_h, t_h, si_v, di_v, w_v,
                rows_v, acc_sh, sem):
    def body(k_i, carry):
        off = (cbase + k_i) * _CH
        pltpu.sync_copy(src_h.at[pl.ds(off, _CH)], si_v)
        pltpu.sync_copy(dst_h.at[pl.ds(off, _CH)], di_v)
        pltpu.sync_copy(w_h.at[pl.ds(off, _CH)], w_v)
        pltpu.async_copy(t_h.at[si_v], rows_v, sem).wait()

        def scale(g, c2):
            wv16 = w_v[pl.ds(g * 16, 16)]
            for i in range(16):
                r = g * 16 + i
                wv = wv16[i]
                for j in range(_H // 16):
                    sl = pl.ds(j * 16, 16)
                    rows_v[r, sl] = rows_v[r, sl] * wv
            return c2

        lax.fori_loop(0, _CH // 16, scale, 0)
        pltpu.sync_copy(rows_v, acc_sh.at[di_v], add=True)
        return carry

    lax.fori_loop(0, ccount, body, 0)


def _spmm_hop1(src, dst, w, t0, t1):
    nchunks = _E // _CH
    per, extra = nchunks // _NT, nchunks % _NT

    @functools.partial(
        pl.kernel,
        mesh=_sc_mesh(),
        out_type=[jax.ShapeDtypeStruct((_NP, _H), jnp.float32),
                  jax.ShapeDtypeStruct((_NP, _H), jnp.float32)],
        scratch_types=[
            pltpu.VMEM((_CH,), jnp.int32),
            pltpu.VMEM((_CH,), jnp.int32),
            pltpu.VMEM((_CH,), jnp.float32),
            pltpu.VMEM((_CH, _H), jnp.float32),
            pltpu.VMEM((_RPT, _H), jnp.float32),
            pltpu.VMEM_SHARED((_NP, _H), jnp.float32),
            pltpu.SemaphoreType.DMA,
        ],
        compiler_params=pltpu.CompilerParams(use_tc_tiling_on_sc=False),
    )
    def k(src_h, dst_h, w_h, t0_h, t1_h, o0_h, o1_h,
          si_v, di_v, w_v, rows_v, stage_v, acc_sh, sem):
        c = lax.axis_index("c")
        s = lax.axis_index("s")
        _zero_stage(stage_v)
        pltpu.sync_copy(stage_v, acc_sh.at[pl.ds(s * _RPT, _RPT)])
        plsc.subcore_barrier()

        cbase = per * s + jnp.minimum(s, extra)
        ccount = per + jnp.where(s < extra, 1, 0)

        @pl.when(c == 0)
        def _():
            _edge_sweep(cbase, ccount, src_h, dst_h, w_h, t0_h,
                        si_v, di_v, w_v, rows_v, acc_sh, sem)

        @pl.when(c == 1)
        def _():
            _edge_sweep(cbase, ccount, src_h, dst_h, w_h, t1_h,
                        si_v, di_v, w_v, rows_v, acc_sh, sem)

        plsc.subcore_barrier()
        r0 = s * _RPT
        pltpu.sync_copy(acc_sh.at[pl.ds(r0, _RPT)], stage_v)

        @pl.when(c == 0)
        def _():
            pltpu.sync_copy(stage_v, o0_h.at[pl.ds(r0, _RPT)])

        @pl.when(c == 1)
        def _():
            pltpu.sync_copy(stage_v, o1_h.at[pl.ds(r0, _RPT)])

    return k(src, dst, w, t0, t1)


def _spmm_hop2(src, dst, w, t):
    nchunks_half = (_E // 2) // _CH
    per, extra = nchunks_half // _NT, nchunks_half % _NT

    @functools.partial(
        pl.kernel,
        mesh=_sc_mesh(),
        out_type=jax.ShapeDtypeStruct((2, _NP, _H), jnp.float32),
        scratch_types=[
            pltpu.VMEM((_CH,), jnp.int32),
            pltpu.VMEM((_CH,), jnp.int32),
            pltpu.VMEM((_CH,), jnp.float32),
            pltpu.VMEM((_CH, _H), jnp.float32),
            pltpu.VMEM((_RPT, _H), jnp.float32),
            pltpu.VMEM_SHARED((_NP, _H), jnp.float32),
            pltpu.SemaphoreType.DMA,
        ],
        compiler_params=pltpu.CompilerParams(use_tc_tiling_on_sc=False),
    )
    def k(src_h, dst_h, w_h, t_h, o_h,
          si_v, di_v, w_v, rows_v, stage_v, acc_sh, sem):
        c = lax.axis_index("c")
        s = lax.axis_index("s")
        _zero_stage(stage_v)
        pltpu.sync_copy(stage_v, acc_sh.at[pl.ds(s * _RPT, _RPT)])
        plsc.subcore_barrier()

        cbase = c * nchunks_half + per * s + jnp.minimum(s, extra)
        ccount = per + jnp.where(s < extra, 1, 0)
        _edge_sweep(cbase, ccount, src_h, dst_h, w_h, t_h,
                    si_v, di_v, w_v, rows_v, acc_sh, sem)

        plsc.subcore_barrier()
        r0 = s * _RPT
        pltpu.sync_copy(acc_sh.at[pl.ds(r0, _RPT)], stage_v)
        pltpu.sync_copy(stage_v, o_h.at[c, pl.ds(r0, _RPT)])

    return k(src, dst, w, t)


def _tc_first(x, wcat):
    def body(x_ref, w_ref, o0_ref, o1_ref):
        u = jnp.dot(x_ref[...], w_ref[...],
                    preferred_element_type=jnp.float32)
        o0_ref[...] = u[:, :_H]
        o1_ref[...] = u[:, _H:]

    return pl.pallas_call(
        body,
        grid=(_N // _ROWBLK,),
        in_specs=[pl.BlockSpec((_ROWBLK, _D), lambda i: (i, 0)),
                  pl.BlockSpec((_D, _D), lambda i: (0, 0))],
        out_specs=[pl.BlockSpec((_ROWBLK, _H), lambda i: (i, 0)),
                   pl.BlockSpec((_ROWBLK, _H), lambda i: (i, 0))],
        out_shape=[jax.ShapeDtypeStruct((_N, _H), jnp.float32),
                   jax.ShapeDtypeStruct((_N, _H), jnp.float32)],
    )(x, wcat)


def _tc_layer(keep, p0, p1, wcat, bvec):
    def body(k_ref, p0_ref, p1_ref, w_ref, b_ref, o0_ref, o1_ref):
        wl = w_ref[...]
        upper = p0_ref[...] + p1_ref[...]
        u = (jnp.dot(k_ref[...], wl[:_H, :],
                     preferred_element_type=jnp.float32)
             + jnp.dot(upper, wl[_H:, :],
                       preferred_element_type=jnp.float32)
             + jnp.dot(b_ref[...], wl, preferred_element_type=jnp.float32))
        o0_ref[...] = u[:, :_H]
        o1_ref[...] = u[:, _H:]

    return pl.pallas_call(
        body,
        grid=(_N // _ROWBLK,),
        in_specs=[pl.BlockSpec((_ROWBLK, _H), lambda i: (i, 0)),
                  pl.BlockSpec((_ROWBLK, _H), lambda i: (i, 0)),
                  pl.BlockSpec((_ROWBLK, _H), lambda i: (i, 0)),
                  pl.BlockSpec((_D, _D), lambda i: (0, 0)),
                  pl.BlockSpec((1, _D), lambda i: (0, 0))],
        out_specs=[pl.BlockSpec((_ROWBLK, _H), lambda i: (i, 0)),
                   pl.BlockSpec((_ROWBLK, _H), lambda i: (i, 0))],
        out_shape=[jax.ShapeDtypeStruct((_N, _H), jnp.float32),
                   jax.ShapeDtypeStruct((_N, _H), jnp.float32)],
    )(keep, p0, p1, wcat, bvec)


def _tc_final(keep, p0, p1, wcp, bvec, bcp):
    def body(k_ref, p0_ref, p1_ref, w_ref, b_ref, bc_ref, o_ref):
        wl = w_ref[...]
        upper = p0_ref[...] + p1_ref[...]
        o_ref[...] = (jnp.dot(k_ref[...], wl[:_H, :],
                              preferred_element_type=jnp.float32)
                      + jnp.dot(upper, wl[_H:, :],
                                preferred_element_type=jnp.float32)
                      + jnp.dot(b_ref[...], wl,
                                preferred_element_type=jnp.float32)
                      + bc_ref[...])

    return pl.pallas_call(
        body,
        grid=(_N // _ROWBLK,),
        in_specs=[pl.BlockSpec((_ROWBLK, _H), lambda i: (i, 0)),
                  pl.BlockSpec((_ROWBLK, _H), lambda i: (i, 0)),
                  pl.BlockSpec((_ROWBLK, _H), lambda i: (i, 0)),
                  pl.BlockSpec((_D, _D), lambda i: (0, 0)),
                  pl.BlockSpec((1, _D), lambda i: (0, 0)),
                  pl.BlockSpec((1, _D), lambda i: (0, 0))],
        out_specs=pl.BlockSpec((_ROWBLK, _D), lambda i: (i, 0)),
        out_shape=jax.ShapeDtypeStruct((_N, _D), jnp.float32),
    )(keep, p0, p1, wcp, bvec, bcp)


def kernel(x, edge_index, edge_weight, W, b, Wc, bc):
    src = edge_index[0]
    dst = edge_index[1]
    w = edge_weight
    nclass = Wc.shape[1]

    t0, t1 = _tc_first(x, jnp.concatenate([W[0, 0], W[0, 1]], axis=1))
    for l in range(W.shape[0]):
        keep, upper = _spmm_hop1(src, dst, w, t0, t1)
        parts = _spmm_hop2(src, dst, w, upper)
        p0, p1 = parts[0], parts[1]
        bvec = jnp.concatenate([b[l, 0], b[l, 1]])[None, :]
        if l + 1 < W.shape[0]:
            wcat = jnp.concatenate([W[l + 1, 0], W[l + 1, 1]], axis=1)
            t0, t1 = _tc_layer(keep, p0, p1, wcat, bvec)
        else:
            wcp = jnp.pad(Wc, ((0, 0), (0, _D - nclass)))
            bcp = jnp.pad(bc, (0, _D - nclass))[None, :]
            out = _tc_final(keep, p0, p1, wcp, bvec, bcp)
            return out[:, :nclass]


# speedup vs baseline: 4.5861x; 1.7860x over previous
import functools

import jax
import jax.numpy as jnp
from jax import lax
from jax.experimental import pallas as pl
from jax.experimental.pallas import tpu as pltpu
from jax.experimental.pallas import tpu_sc as plsc

_N = 10000
_E = 320000
_EP = 327680
_D = 128
_H = 64
_CH = 128
_SBC = 4
_SB = _SBC * _CH
_NT = 16
_NP = 10240
_RPT = _NP // _NT
_NCHUNK = _EP // _CH
_ROWBLK = 1000


def _sc_mesh():
    return plsc.VectorSubcoreMesh(core_axis_name="c", subcore_axis_name="s")


def _zero_buf(buf, nrows):
    zero16 = jnp.zeros((16,), jnp.float32)

    def zrow(r, carry):
        for j in range(_H // 16):
            buf[r, pl.ds(j * 16, 16)] = zero16
        return carry

    lax.fori_loop(0, nrows, zrow, 0)


def _sc_scratch(cpt):
    del cpt
    bufs = []
    for _ in range(2):
        bufs += [pltpu.VMEM((_SBC, _CH), jnp.int32),
                 pltpu.VMEM((_SBC, _CH), jnp.int32),
                 pltpu.VMEM((_SBC, _CH), jnp.float32),
                 pltpu.VMEM((_SB, _H), jnp.float32)]
    return bufs + [
        pltpu.VMEM_SHARED((_NP, _H), jnp.float32),
        pltpu.SemaphoreType.DMA,
        pltpu.SemaphoreType.DMA,
        pltpu.SemaphoreType.DMA,
        pltpu.SemaphoreType.DMA,
        pltpu.SemaphoreType.DMA,
        pltpu.SemaphoreType.DMA,
    ]


def _pipeline(slab0, nsb, bufs, acc_sh, t_h, src2_h, dst2_h, w2_h):
    def load_idx(hb, sbi, sem):
        src_v, dst_v, w_v = hb[0], hb[1], hb[2]
        row = slab0 + sbi * _SBC
        pltpu.async_copy(src2_h.at[pl.ds(row, _SBC)], src_v, sem)
        pltpu.async_copy(dst2_h.at[pl.ds(row, _SBC)], dst_v, sem)
        pltpu.async_copy(w2_h.at[pl.ds(row, _SBC)], w_v, sem)

    def wait_idx(hb, sem):
        src_v, dst_v, w_v = hb[0], hb[1], hb[2]
        row = slab0
        pltpu.make_async_copy(src2_h.at[pl.ds(row, _SBC)], src_v, sem).wait()
        pltpu.make_async_copy(dst2_h.at[pl.ds(row, _SBC)], dst_v, sem).wait()
        pltpu.make_async_copy(w2_h.at[pl.ds(row, _SBC)], w_v, sem).wait()

    def fire_gathers(hb):
        src_v, rows_v, sem = hb[0], hb[3], hb[6]
        for j in range(_SBC):
            pltpu.async_copy(t_h.at[src_v.at[j]],
                             rows_v.at[pl.ds(j * _CH, _CH)], sem)

    def drain_gathers(hb):
        src_v, rows_v, sem = hb[0], hb[3], hb[6]
        for j in range(_SBC):
            pltpu.make_async_copy(t_h.at[src_v.at[j]],
                                  rows_v.at[pl.ds(j * _CH, _CH)], sem).wait()

    def scale_scatter(hb):
        dst_v, w_v, rows_v, sem = hb[1], hb[2], hb[3], hb[7]
        for j in range(_SBC):

            def grp(g, carry, j=j):
                wv16 = w_v[j, pl.ds(g * 16, 16)]
                for i in range(16):
                    r = j * _CH + g * 16 + i
                    wv = wv16[i]
                    for q in range(_H // 16):
                        sl = pl.ds(q * 16, 16)
                        rows_v[r, sl] = rows_v[r, sl] * wv
                return carry

            lax.fori_loop(0, _CH // 16, grp, 0)
            pltpu.async_copy(rows_v.at[pl.ds(j * _CH, _CH)],
                             acc_sh.at[dst_v.at[j]], sem, add=True)

    def drain_scatters(hb):
        dst_v, rows_v, sem = hb[1], hb[3], hb[7]
        for j in range(_SBC):
            pltpu.make_async_copy(rows_v.at[pl.ds(j * _CH, _CH)],
                                  acc_sh.at[dst_v.at[j]], sem).wait()

    buf_a, buf_b = bufs
    npairs = nsb // 2

    load_idx(buf_a, 0, buf_a[5])
    load_idx(buf_b, 1, buf_b[5])
    wait_idx(buf_a, buf_a[5])
    fire_gathers(buf_a)
    wait_idx(buf_b, buf_b[5])
    fire_gathers(buf_b)

    def half(hb, sb_next, is_not_last):
        drain_gathers(hb)
        scale_scatter(hb)
        drain_scatters(hb)

        @pl.when(is_not_last)
        def _():
            load_idx(hb, sb_next, hb[5])
            wait_idx(hb, hb[5])
            fire_gathers(hb)

    def pair(pi, carry):
        not_last = pi < npairs - 1
        half(buf_a, pi * 2 + 2, not_last)
        half(buf_b, pi * 2 + 3, not_last)
        return carry

    lax.fori_loop(0, npairs, pair, 0)


def _zero_acc(rows_a, acc_sh, s):
    _zero_buf(rows_a, _SB)
    pltpu.sync_copy(rows_a, acc_sh.at[pl.ds(s * _RPT, _SB)])
    pltpu.sync_copy(rows_a.at[pl.ds(0, _RPT - _SB)],
                    acc_sh.at[pl.ds(s * _RPT + _SB, _RPT - _SB)])


def _write_out(rows_a, acc_sh, o_slice, s):
    r0 = s * _RPT
    pltpu.sync_copy(acc_sh.at[pl.ds(r0, _SB)], rows_a)
    pltpu.sync_copy(rows_a, o_slice.at[pl.ds(r0, _SB)])
    pltpu.sync_copy(acc_sh.at[pl.ds(r0 + _SB, _RPT - _SB)],
                    rows_a.at[pl.ds(0, _RPT - _SB)])
    pltpu.sync_copy(rows_a.at[pl.ds(0, _RPT - _SB)],
                    o_slice.at[pl.ds(r0 + _SB, _RPT - _SB)])


def _spmm_hop1(src2, dst2, w2, t0, t1):
    cpt = _NCHUNK // _NT
    nsb = cpt // _SBC

    @functools.partial(
        pl.kernel,
        mesh=_sc_mesh(),
        out_type=[jax.ShapeDtypeStruct((_NP, _H), jnp.float32),
                  jax.ShapeDtypeStruct((_NP, _H), jnp.float32)],
        scratch_types=_sc_scratch(cpt),
        compiler_params=pltpu.CompilerParams(use_tc_tiling_on_sc=False),
    )
    def k(src2_h, dst2_h, w2_h, t0_h, t1_h, o0_h, o1_h,
          src_a, dst_a, w_a, rows_a, src_b, dst_b, w_b, rows_b, acc_sh,
          sida, sidb, sga, sgb, ssa, ssb):
        c = lax.axis_index("c")
        s = lax.axis_index("s")
        buf_a = (src_a, dst_a, w_a, rows_a, None, sida, sga, ssa)
        buf_b = (src_b, dst_b, w_b, rows_b, None, sidb, sgb, ssb)
        _zero_acc(rows_a, acc_sh, s)
        plsc.subcore_barrier()

        slab0 = s * cpt

        @pl.when(c == 0)
        def _():
            _pipeline(slab0, nsb, (buf_a, buf_b), acc_sh, t0_h,
                      src2_h, dst2_h, w2_h)

        @pl.when(c == 1)
        def _():
            _pipeline(slab0, nsb, (buf_a, buf_b), acc_sh, t1_h,
                      src2_h, dst2_h, w2_h)

        plsc.subcore_barrier()

        @pl.when(c == 0)
        def _():
            _write_out(rows_a, acc_sh, o0_h, s)

        @pl.when(c == 1)
        def _():
            _write_out(rows_a, acc_sh, o1_h, s)

    return k(src2, dst2, w2, t0, t1)


def _spmm_hop2(src2, dst2, w2, t):
    half = _NCHUNK // 2
    cpt = half // _NT
    nsb = cpt // _SBC

    @functools.partial(
        pl.kernel,
        mesh=_sc_mesh(),
        out_type=jax.ShapeDtypeStruct((2, _NP, _H), jnp.float32),
        scratch_types=_sc_scratch(cpt),
        compiler_params=pltpu.CompilerParams(use_tc_tiling_on_sc=False),
    )
    def k(src2_h, dst2_h, w2_h, t_h, o_h,
          src_a, dst_a, w_a, rows_a, src_b, dst_b, w_b, rows_b, acc_sh,
          sida, sidb, sga, sgb, ssa, ssb):
        c = lax.axis_index("c")
        s = lax.axis_index("s")
        buf_a = (src_a, dst_a, w_a, rows_a, None, sida, sga, ssa)
        buf_b = (src_b, dst_b, w_b, rows_b, None, sidb, sgb, ssb)
        _zero_acc(rows_a, acc_sh, s)
        plsc.subcore_barrier()

        slab0 = c * half + s * cpt
        _pipeline(slab0, nsb, (buf_a, buf_b), acc_sh, t_h,
                  src2_h, dst2_h, w2_h)

        plsc.subcore_barrier()
        _write_out(rows_a, acc_sh, o_h.at[c], s)

    return k(src2, dst2, w2, t)


def _tc_first(x, wcat):
    def body(x_ref, w_ref, o0_ref, o1_ref):
        u = jnp.dot(x_ref[...], w_ref[...],
                    preferred_element_type=jnp.float32)
        o0_ref[...] = u[:, :_H]
        o1_ref[...] = u[:, _H:]

    return pl.pallas_call(
        body,
        grid=(_N // _ROWBLK,),
        in_specs=[pl.BlockSpec((_ROWBLK, _D), lambda i: (i, 0)),
                  pl.BlockSpec((_D, _D), lambda i: (0, 0))],
        out_specs=[pl.BlockSpec((_ROWBLK, _H), lambda i: (i, 0)),
                   pl.BlockSpec((_ROWBLK, _H), lambda i: (i, 0))],
        out_shape=[jax.ShapeDtypeStruct((_N, _H), jnp.float32),
                   jax.ShapeDtypeStruct((_N, _H), jnp.float32)],
    )(x, wcat)


def _tc_layer(keep, p0, p1, wcat, bvec):
    def body(k_ref, p0_ref, p1_ref, w_ref, b_ref, o0_ref, o1_ref):
        wl = w_ref[...]
        upper = p0_ref[...] + p1_ref[...]
        u = (jnp.dot(k_ref[...], wl[:_H, :],
                     preferred_element_type=jnp.float32)
             + jnp.dot(upper, wl[_H:, :],
                       preferred_element_type=jnp.float32)
             + jnp.dot(b_ref[...], wl, preferred_element_type=jnp.float32))
        o0_ref[...] = u[:, :_H]
        o1_ref[...] = u[:, _H:]

    return pl.pallas_call(
        body,
        grid=(_N // _ROWBLK,),
        in_specs=[pl.BlockSpec((_ROWBLK, _H), lambda i: (i, 0)),
                  pl.BlockSpec((_ROWBLK, _H), lambda i: (i, 0)),
                  pl.BlockSpec((_ROWBLK, _H), lambda i: (i, 0)),
                  pl.BlockSpec((_D, _D), lambda i: (0, 0)),
                  pl.BlockSpec((1, _D), lambda i: (0, 0))],
        out_specs=[pl.BlockSpec((_ROWBLK, _H), lambda i: (i, 0)),
                   pl.BlockSpec((_ROWBLK, _H), lambda i: (i, 0))],
        out_shape=[jax.ShapeDtypeStruct((_N, _H), jnp.float32),
                   jax.ShapeDtypeStruct((_N, _H), jnp.float32)],
    )(keep, p0, p1, wcat, bvec)


def _tc_final(keep, p0, p1, wcp, bvec, bcp):
    def body(k_ref, p0_ref, p1_ref, w_ref, b_ref, bc_ref, o_ref):
        wl = w_ref[...]
        upper = p0_ref[...] + p1_ref[...]
        o_ref[...] = (jnp.dot(k_ref[...], wl[:_H, :],
                              preferred_element_type=jnp.float32)
                      + jnp.dot(upper, wl[_H:, :],
                                preferred_element_type=jnp.float32)
                      + jnp.dot(b_ref[...], wl,
                                preferred_element_type=jnp.float32)
                      + bc_ref[...])

    return pl.pallas_call(
        body,
        grid=(_N // _ROWBLK,),
        in_specs=[pl.BlockSpec((_ROWBLK, _H), lambda i: (i, 0)),
                  pl.BlockSpec((_ROWBLK, _H), lambda i: (i, 0)),
                  pl.BlockSpec((_ROWBLK, _H), lambda i: (i, 0)),
                  pl.BlockSpec((_D, _D), lambda i: (0, 0)),
                  pl.BlockSpec((1, _D), lambda i: (0, 0)),
                  pl.BlockSpec((1, _D), lambda i: (0, 0))],
        out_specs=pl.BlockSpec((_ROWBLK, _D), lambda i: (i, 0)),
        out_shape=jax.ShapeDtypeStruct((_N, _D), jnp.float32),
    )(keep, p0, p1, wcp, bvec, bcp)


def kernel(x, edge_index, edge_weight, W, b, Wc, bc):
    pad = _EP - _E
    src2 = jnp.concatenate(
        [edge_index[0], jnp.zeros((pad,), jnp.int32)]).reshape(_NCHUNK, _CH)
    dst2 = jnp.concatenate(
        [edge_index[1], jnp.zeros((pad,), jnp.int32)]).reshape(_NCHUNK, _CH)
    w2 = jnp.concatenate(
        [edge_weight, jnp.zeros((pad,), jnp.float32)]).reshape(_NCHUNK, _CH)
    nclass = Wc.shape[1]

    t0, t1 = _tc_first(x, jnp.concatenate([W[0, 0], W[0, 1]], axis=1))
    for l in range(W.shape[0]):
        keep, upper = _spmm_hop1(src2, dst2, w2, t0, t1)
        parts = _spmm_hop2(src2, dst2, w2, upper)
        p0, p1 = parts[0], parts[1]
        bvec = jnp.concatenate([b[l, 0], b[l, 1]])[None, :]
        if l + 1 < W.shape[0]:
            wcat = jnp.concatenate([W[l + 1, 0], W[l + 1, 1]], axis=1)
            t0, t1 = _tc_layer(keep, p0, p1, wcat, bvec)
        else:
            wcp = jnp.pad(Wc, ((0, 0), (0, _D - nclass)))
            bcp = jnp.pad(bc, (0, _D - nclass))[None, :]
            out = _tc_final(keep, p0, p1, wcp, bvec, bcp)
            return out[:, :nclass]
